# Initial kernel scaffold; baseline (speedup 1.0000x reference)
import jax
import jax.numpy as jnp
from jax.experimental import pallas as pl


def kernel(x, target):
    raise NotImplementedError("write your pallas kernel here")


# speedup vs baseline: 6.6007x; 6.6007x over previous
import math

import jax
import jax.numpy as jnp
from jax.experimental import pallas as pl
from jax.experimental.pallas import tpu as pltpu

_SIZE = 32000
_SMOOTHING = 0.1
_CONF = 1.0 - _SMOOTHING
_EPS = _SMOOTHING / (_SIZE - 2)
_PAD = 0
_C_ROW = (_SIZE - 2) * _EPS * math.log(_EPS) + _CONF * math.log(_CONF)

_BR = 256
_BC = 6400


def _loss_tile(t_ref, x_ref, out_ref):
    r = pl.program_id(0)
    c = pl.program_id(1)

    @pl.when((r == 0) & (c == 0))
    def _init():
        out_ref[0, 0] = 0.0

    x = x_ref[...]
    t = t_ref[0]
    col = jax.lax.broadcasted_iota(jnp.int32, (_BR, _BC), 1) + c * _BC
    w = jnp.where(col == t, -_CONF, -_EPS)
    w = jnp.where(col == _PAD, 0.0, w)
    w = jnp.where(t == _PAD, 0.0, w)
    partial = jnp.sum(x * w)

    nonpad = jnp.sum((t != _PAD).astype(jnp.float32))
    partial = partial + jnp.where(c == 0, _C_ROW * nonpad, 0.0)

    out_ref[0, 0] += partial


def kernel(x, target):
    N, V = x.shape
    assert V == _SIZE and N % _BR == 0 and V % _BC == 0
    nr, nc = N // _BR, V // _BC
    t3 = target.astype(jnp.int32).reshape(nr, _BR, 1)
    out = pl.pallas_call(
        _loss_tile,
        grid=(nr, nc),
        in_specs=[
            pl.BlockSpec((1, _BR, 1), lambda r, c: (r, 0, 0)),
            pl.BlockSpec((_BR, _BC), lambda r, c: (r, c)),
        ],
        out_specs=pl.BlockSpec(
            (1, 1), lambda r, c: (0, 0), memory_space=pltpu.SMEM
        ),
        out_shape=jax.ShapeDtypeStruct((1, 1), jnp.float32),
    )(t3, x)
    return out[0, 0]
